# Initial kernel scaffold; baseline (speedup 1.0000x reference)
import jax
import jax.numpy as jnp
from jax.experimental import pallas as pl


def kernel(x, edge_index, W1, b1, W2, b2, Wp1, bp1, Wp2, bp2):
    raise NotImplementedError("write your pallas kernel here")


# speedup vs baseline: 11.8219x; 11.8219x over previous
import functools

import jax
import jax.numpy as jnp
from jax import lax
from jax.experimental import pallas as pl
from jax.experimental.pallas import tpu as pltpu
from jax.experimental.pallas import tpu_sc as plsc

NC = 2
NS = 16
NW = NC * NS
LN = 16

_SC_PARAMS = pltpu.CompilerParams(
    needs_layout_passes=False, use_tc_tiling_on_sc=False)

_MESH = dict(core_axis_name="c", subcore_axis_name="s")


def _sc_deg(dst, n_pad, e):
    ew = e // NW
    cd = 2000
    assert ew % cd == 0 and n_pad % LN == 0

    @functools.partial(
        pl.kernel,
        out_type=jax.ShapeDtypeStruct((NW, n_pad), jnp.float32),
        mesh=plsc.VectorSubcoreMesh(**_MESH),
        compiler_params=_SC_PARAMS,
        scratch_types=[
            pltpu.VMEM((n_pad,), jnp.float32),
            pltpu.VMEM((cd,), jnp.int32),
        ],
    )
    def k(dst_h, out_h, deg_v, idx_v):
        c = lax.axis_index("c")
        s = lax.axis_index("s")
        w = s * NC + c
        z = jnp.zeros((LN,), jnp.float32)

        @pl.loop(0, n_pad // LN, unroll=8)
        def _(i):
            deg_v[pl.ds(i * LN, LN)] = z

        ones = jnp.ones((LN,), jnp.float32)
        base0 = w * ew

        @pl.loop(0, ew // cd)
        def _(j):
            pltpu.sync_copy(dst_h.at[pl.ds(base0 + j * cd, cd)], idx_v)

            @pl.loop(0, cd // LN, unroll=8)
            def _(t):
                idx = idx_v[pl.ds(t * LN, LN)]
                plsc.addupdate_scatter(deg_v, [idx], ones)

        pltpu.sync_copy(deg_v, out_h.at[w])

    return k(dst)


def _sc_agg8(src, dst, xs8, z8, n_pad, e):
    ew = e // NW
    ce = 2000
    zc = 512
    nch = n_pad // zc
    per = -(-nch // NS)
    assert ew % ce == 0 and n_pad % zc == 0

    @functools.partial(
        pl.kernel,
        out_type=jax.ShapeDtypeStruct((NC, n_pad, 8), jnp.float32),
        mesh=plsc.VectorSubcoreMesh(**_MESH),
        compiler_params=_SC_PARAMS,
        scratch_types=[
            pltpu.VMEM((ce,), jnp.int32),
            pltpu.VMEM((ce,), jnp.int32),
            pltpu.VMEM((ce, 8), jnp.float32),
            pltpu.VMEM((zc, 8), jnp.float32),
            pltpu.VMEM_SHARED((n_pad, 8), jnp.float32),
            pltpu.SemaphoreType.DMA,
        ],
    )
    def k(src_h, dst_h, xs_h, z8_h, out_h, src_v, dst_v, rows_v, buf_v,
          acc_sh, sem):
        c = lax.axis_index("c")
        s = lax.axis_index("s")
        w = s * NC + c

        pltpu.sync_copy(z8_h, buf_v)

        @pl.loop(0, per)
        def _(j):
            ci = s * per + j

            @pl.when(ci < nch)
            def _():
                pltpu.sync_copy(buf_v, acc_sh.at[pl.ds(ci * zc, zc)])

        plsc.subcore_barrier()

        base0 = w * ew

        @pl.loop(0, ew // ce)
        def _(j):
            base = base0 + j * ce
            pltpu.sync_copy(src_h.at[pl.ds(base, ce)], src_v)
            pltpu.sync_copy(dst_h.at[pl.ds(base, ce)], dst_v)
            pltpu.async_copy(xs_h.at[src_v], rows_v, sem).wait()
            pltpu.sync_copy(rows_v, acc_sh.at[dst_v], add=True)

        plsc.subcore_barrier()

        @pl.loop(0, per)
        def _(j):
            ci = s * per + j

            @pl.when(ci < nch)
            def _():
                pltpu.sync_copy(acc_sh.at[pl.ds(ci * zc, zc)], buf_v)
                pltpu.sync_copy(buf_v, out_h.at[c, pl.ds(ci * zc, zc)])

    return k(src, dst, xs8, z8)


def _sc_agg64(src, dst, hs, zrows, n_pad, e):
    npart = 10
    rchunk = 10240
    assert npart * rchunk >= n_pad
    ept = e // NS
    cs = 2000
    g = 6144
    flush_at = 4096
    sub = 512
    nzc = rchunk // 512
    per = -(-nzc // NS)
    assert ept % cs == 0 and rchunk % 512 == 0

    @functools.partial(
        pl.kernel,
        out_type=jax.ShapeDtypeStruct((npart * rchunk, 64), jnp.float32),
        mesh=plsc.VectorSubcoreMesh(**_MESH),
        compiler_params=_SC_PARAMS,
        scratch_types=[
            pltpu.VMEM((cs,), jnp.int32),
            pltpu.VMEM((cs,), jnp.int32),
            pltpu.VMEM((g + sub,), jnp.int32),
            pltpu.VMEM((g + sub,), jnp.int32),
            pltpu.VMEM((sub, 64), jnp.float32),
            pltpu.VMEM((512, 64), jnp.float32),
            pltpu.VMEM_SHARED((rchunk + 512, 64), jnp.float32),
            pltpu.SemaphoreType.DMA,
        ],
    )
    def k(src_h, dst_h, hs_h, zr_h, out_h, srcb, dstb, csrc, cdst, rows_v,
          zbuf_v, acc_sh, sem):
        c = lax.axis_index("c")
        s = lax.axis_index("s")
        trash = jnp.int32(rchunk + s * 16)
        zsent = jnp.zeros((LN,), jnp.int32)
        tsent = jnp.full((LN,), 1, jnp.int32) * trash

        pltpu.sync_copy(zr_h, zbuf_v)

        def flush(cnt):
            for t in range(sub // LN):
                csrc[pl.ds(cnt + t * LN, LN)] = zsent
                cdst[pl.ds(cnt + t * LN, LN)] = tsent
            nr = (cnt + (sub - 1)) // sub

            @pl.loop(0, nr)
            def _(r):
                pltpu.async_copy(
                    hs_h.at[csrc.at[pl.ds(r * sub, sub)]], rows_v, sem
                ).wait()
                pltpu.sync_copy(
                    rows_v, acc_sh.at[cdst.at[pl.ds(r * sub, sub)]],
                    add=True)

            return jnp.int32(0)

        for p in range(npart // NC):
            chunk_id = c * (npart // NC) + p
            lo = chunk_id * rchunk
            hi = lo + rchunk

            @pl.loop(0, per)
            def _(j):
                ci = s * per + j

                @pl.when(ci < nzc)
                def _():
                    pltpu.sync_copy(zbuf_v, acc_sh.at[pl.ds(ci * 512, 512)])

            plsc.subcore_barrier()

            base0 = s * ept

            def chunk_body(j, cnt):
                base = base0 + j * cs
                pltpu.sync_copy(src_h.at[pl.ds(base, cs)], srcb)
                pltpu.sync_copy(dst_h.at[pl.ds(base, cs)], dstb)

                def vreg_body(t, cnt):
                    d = dstb[pl.ds(t * LN, LN)]
                    sv = srcb[pl.ds(t * LN, LN)]
                    m = (d >= lo) & (d < hi)
                    plsc.store_compressed(cdst.at[pl.ds(cnt, LN)], d - lo,
                                          mask=m)
                    plsc.store_compressed(csrc.at[pl.ds(cnt, LN)], sv,
                                          mask=m)
                    return cnt + jnp.sum(jnp.where(m, 1, 0)).astype(
                        jnp.int32)

                cnt = pl.loop(0, cs // LN, init_carry=cnt, unroll=4)(
                    vreg_body)
                return lax.cond(cnt >= flush_at, flush, lambda x: x, cnt)

            cnt = pl.loop(0, ept // cs, init_carry=jnp.int32(0))(chunk_body)
            lax.cond(cnt > 0, flush, lambda x: x, cnt)

            plsc.subcore_barrier()

            @pl.loop(0, per)
            def _(j):
                ci = s * per + j

                @pl.when(ci < nzc)
                def _():
                    pltpu.sync_copy(acc_sh.at[pl.ds(ci * 512, 512)], rows_v)
                    pltpu.sync_copy(
                        rows_v, out_h.at[pl.ds(lo + ci * 512, 512)])

            plsc.subcore_barrier()

    return k(src, dst, hs, zrows)


def _tc_prep(deg_parts, x_pad, n, n_pad):
    blk = 256
    grid = n_pad // blk

    def body(deg_ref, x_ref, dinv_ref, xs_ref, gs_ref):
        i = pl.program_id(0)
        deg = jnp.sum(deg_ref[...], axis=0) + 1.0
        dinv = lax.rsqrt(deg)
        dinv2 = dinv.reshape(blk, 1)
        dinv_ref[...] = dinv2
        xb = x_ref[...]
        xs_ref[...] = xb * dinv2
        m = (xb[:, 2:3] == 1.0).astype(jnp.float32)
        terms = jnp.concatenate(
            [xb[:, 0:2] * m, xb[:, 2:5], m, jnp.zeros((blk, 2),
                                                      jnp.float32)],
            axis=1)
        gsum = jnp.sum(terms, axis=0).reshape(1, 8)

        @pl.when(i == 0)
        def _():
            gs_ref[...] = jnp.zeros((1, 8), jnp.float32)

        gs_ref[...] += gsum

    return pl.pallas_call(
        body,
        grid=(grid,),
        in_specs=[
            pl.BlockSpec((NW, blk), lambda i: (0, i)),
            pl.BlockSpec((blk, 8), lambda i: (i, 0)),
        ],
        out_specs=[
            pl.BlockSpec((blk, 1), lambda i: (i, 0)),
            pl.BlockSpec((blk, 8), lambda i: (i, 0)),
            pl.BlockSpec((1, 8), lambda i: (0, 0)),
        ],
        out_shape=[
            jax.ShapeDtypeStruct((n_pad, 1), jnp.float32),
            jax.ShapeDtypeStruct((n_pad, 8), jnp.float32),
            jax.ShapeDtypeStruct((1, 8), jnp.float32),
        ],
    )(deg_parts, x_pad)


def _tc_layer1(agg8, xs8, dinv, w1p, b1r, n_pad):
    blk = 256
    grid = n_pad // blk

    def body(ag_ref, xs_ref, dinv_ref, w_ref, b_ref, hs_ref):
        sagg = ag_ref[0] + ag_ref[1] + xs_ref[...]
        h = jnp.dot(sagg, w_ref[...], preferred_element_type=jnp.float32)
        dinv2 = dinv_ref[...]
        h = jnp.maximum(h * dinv2 + b_ref[...], 0.0)
        hs_ref[...] = h * dinv2

    return pl.pallas_call(
        body,
        grid=(grid,),
        in_specs=[
            pl.BlockSpec((NC, blk, 8), lambda i: (0, i, 0)),
            pl.BlockSpec((blk, 8), lambda i: (i, 0)),
            pl.BlockSpec((blk, 1), lambda i: (i, 0)),
            pl.BlockSpec((8, 64), lambda i: (0, 0)),
            pl.BlockSpec((1, 64), lambda i: (0, 0)),
        ],
        out_specs=pl.BlockSpec((blk, 64), lambda i: (i, 0)),
        out_shape=jax.ShapeDtypeStruct((n_pad, 64), jnp.float32),
    )(agg8, xs8, dinv, w1p, b1r)


def _tc_final(agg64, hs, dinv, w2, b2r, gf8, wp1a, wp1b, bp1r, wp2, bp2r,
              n, n_pad):
    blk = 256
    grid = n_pad // blk

    def body(ag_ref, hs_ref, dinv_ref, w2_ref, b2_ref, gf_ref, wa_ref,
             wb_ref, b3_ref, wc_ref, b4_ref, out_ref, acc_ref):
        i = pl.program_id(0)
        sagg = ag_ref[...] + hs_ref[...]
        h = jnp.dot(sagg, w2_ref[...], preferred_element_type=jnp.float32)
        h = jnp.maximum(h * dinv_ref[...] + b2_ref[...], 0.0)
        rid = lax.broadcasted_iota(jnp.int32, (blk, 1), 0) + i * blk
        h = jnp.where(rid < n, h, 0.0)

        @pl.when(i == 0)
        def _():
            acc_ref[...] = jnp.zeros((1, 64), jnp.float32)

        acc_ref[...] += jnp.sum(h, axis=0).reshape(1, 64)

        @pl.when(i == grid - 1)
        def _():
            emb = acc_ref[...] * (1.0 / n)
            hid = jnp.dot(emb, wa_ref[...],
                          preferred_element_type=jnp.float32)
            hid += jnp.dot(gf_ref[...], wb_ref[...],
                           preferred_element_type=jnp.float32)
            hid = jnp.maximum(hid + b3_ref[...], 0.0)
            raw = jnp.dot(hid, wc_ref[...],
                          preferred_element_type=jnp.float32) + b4_ref[...]
            sig = 1.0 / (1.0 + jnp.exp(-raw))
            out_ref[...] = 2.0 + sig * 3.0

    return pl.pallas_call(
        body,
        grid=(grid,),
        in_specs=[
            pl.BlockSpec((blk, 64), lambda i: (i, 0)),
            pl.BlockSpec((blk, 64), lambda i: (i, 0)),
            pl.BlockSpec((blk, 1), lambda i: (i, 0)),
            pl.BlockSpec((64, 64), lambda i: (0, 0)),
            pl.BlockSpec((1, 64), lambda i: (0, 0)),
            pl.BlockSpec((1, 8), lambda i: (0, 0)),
            pl.BlockSpec((64, 32), lambda i: (0, 0)),
            pl.BlockSpec((8, 32), lambda i: (0, 0)),
            pl.BlockSpec((1, 32), lambda i: (0, 0)),
            pl.BlockSpec((32, 2), lambda i: (0, 0)),
            pl.BlockSpec((1, 2), lambda i: (0, 0)),
        ],
        out_specs=pl.BlockSpec((1, 2), lambda i: (0, 0)),
        out_shape=jax.ShapeDtypeStruct((1, 2), jnp.float32),
        scratch_shapes=[pltpu.VMEM((1, 64), jnp.float32)],
    )(agg64, hs, dinv, w2, b2r, gf8, wp1a, wp1b, bp1r, wp2, bp2r)


def kernel(x, edge_index, W1, b1, W2, b2, Wp1, bp1, Wp2, bp2):
    n, f = x.shape
    e = edge_index.shape[1]
    n_pad = -(-n // 512) * 512
    h = W1.shape[1]
    assert h == 64 and f == 5

    e32 = edge_index.astype(jnp.int32)
    src, dst = e32[0], e32[1]

    x_pad = jnp.zeros((n_pad, 8), jnp.float32).at[:n, :f].set(x)
    w1p = jnp.zeros((8, h), jnp.float32).at[:f].set(W1)
    b1r = b1.reshape(1, h)
    b2r = b2.reshape(1, h)
    wp1a = Wp1[:h]
    wp1b = jnp.zeros((8, 32), jnp.float32).at[:6].set(Wp1[h:])
    bp1r = bp1.reshape(1, 32)
    bp2r = bp2.reshape(1, 2)
    z8 = jnp.zeros((512, 8), jnp.float32)
    zrows = jnp.zeros((512, 64), jnp.float32)

    deg_parts = _sc_deg(dst, n_pad, e)
    dinv, xs8, gsums = _tc_prep(deg_parts, x_pad, n, n_pad)

    s0m, s1m, s2, s3, s4, cntm = (gsums[0, j] for j in range(6))
    avg_l = jnp.where(cntm > 0, s0m / jnp.maximum(cntm, 1.0), 0.0)
    avg_m = jnp.where(cntm > 0, s1m / jnp.maximum(cntm, 1.0), 0.0)
    gf8 = jnp.stack([s2, s3, s4, s3 + s4, avg_l, avg_m,
                     jnp.float32(0), jnp.float32(0)]).reshape(1, 8)

    agg8 = _sc_agg8(src, dst, xs8, z8, n_pad, e)
    hs = _tc_layer1(agg8, xs8, dinv, w1p, b1r, n_pad)
    agg64 = _sc_agg64(src, dst, hs, zrows, n_pad, e)
    return _tc_final(agg64, hs, dinv, W2, b2r, gf8, wp1a, wp1b, bp1r, Wp2,
                     bp2r, n, n_pad)


# speedup vs baseline: 21.9648x; 1.8580x over previous
import functools

import jax
import jax.numpy as jnp
from jax import lax
from jax.experimental import pallas as pl
from jax.experimental.pallas import tpu as pltpu
from jax.experimental.pallas import tpu_sc as plsc

NC = 2
NS = 16
NW = NC * NS
LN = 16

_SC_PARAMS = pltpu.CompilerParams(
    needs_layout_passes=False, use_tc_tiling_on_sc=False)

_MESH = dict(core_axis_name="c", subcore_axis_name="s")


def _sc_deg(dst, n_pad, e):
    ew = e // NW
    cd = 2000
    assert ew % cd == 0 and n_pad % LN == 0

    @functools.partial(
        pl.kernel,
        out_type=jax.ShapeDtypeStruct((NW, n_pad), jnp.float32),
        mesh=plsc.VectorSubcoreMesh(**_MESH),
        compiler_params=_SC_PARAMS,
        scratch_types=[
            pltpu.VMEM((n_pad,), jnp.float32),
            pltpu.VMEM((cd,), jnp.int32),
        ],
    )
    def k(dst_h, out_h, deg_v, idx_v):
        c = lax.axis_index("c")
        s = lax.axis_index("s")
        w = s * NC + c
        z = jnp.zeros((LN,), jnp.float32)

        @pl.loop(0, n_pad // LN, unroll=8)
        def _(i):
            deg_v[pl.ds(i * LN, LN)] = z

        ones = jnp.ones((LN,), jnp.float32)
        base0 = w * ew

        @pl.loop(0, ew // cd)
        def _(j):
            pltpu.sync_copy(dst_h.at[pl.ds(base0 + j * cd, cd)], idx_v)

            @pl.loop(0, cd // LN, unroll=8)
            def _(t):
                idx = idx_v[pl.ds(t * LN, LN)]
                plsc.addupdate_scatter(deg_v, [idx], ones)

        pltpu.sync_copy(deg_v, out_h.at[w])

    return k(dst)


def _sc_agg8(src, dst, xs8, z8, n_pad, e):
    ew = e // NW
    ce = 2000
    zc = 512
    nch = n_pad // zc
    per = -(-nch // NS)
    assert ew % ce == 0 and n_pad % zc == 0

    @functools.partial(
        pl.kernel,
        out_type=jax.ShapeDtypeStruct((NC, n_pad, 8), jnp.float32),
        mesh=plsc.VectorSubcoreMesh(**_MESH),
        compiler_params=_SC_PARAMS,
        scratch_types=[
            pltpu.VMEM((ce,), jnp.int32),
            pltpu.VMEM((ce,), jnp.int32),
            pltpu.VMEM((ce, 8), jnp.float32),
            pltpu.VMEM((zc, 8), jnp.float32),
            pltpu.VMEM_SHARED((n_pad, 8), jnp.float32),
            pltpu.SemaphoreType.DMA,
        ],
    )
    def k(src_h, dst_h, xs_h, z8_h, out_h, src_v, dst_v, rows_v, buf_v,
          acc_sh, sem):
        c = lax.axis_index("c")
        s = lax.axis_index("s")
        w = s * NC + c

        pltpu.sync_copy(z8_h, buf_v)

        @pl.loop(0, per)
        def _(j):
            ci = s * per + j

            @pl.when(ci < nch)
            def _():
                pltpu.sync_copy(buf_v, acc_sh.at[pl.ds(ci * zc, zc)])

        plsc.subcore_barrier()

        base0 = w * ew

        @pl.loop(0, ew // ce)
        def _(j):
            base = base0 + j * ce
            pltpu.sync_copy(src_h.at[pl.ds(base, ce)], src_v)
            pltpu.sync_copy(dst_h.at[pl.ds(base, ce)], dst_v)
            pltpu.async_copy(xs_h.at[src_v], rows_v, sem).wait()
            pltpu.sync_copy(rows_v, acc_sh.at[dst_v], add=True)

        plsc.subcore_barrier()

        @pl.loop(0, per)
        def _(j):
            ci = s * per + j

            @pl.when(ci < nch)
            def _():
                pltpu.sync_copy(acc_sh.at[pl.ds(ci * zc, zc)], buf_v)
                pltpu.sync_copy(buf_v, out_h.at[c, pl.ds(ci * zc, zc)])

    return k(src, dst, xs8, z8)


def _sc_agg64(src, dst, hs, zrows, n_pad, e):
    npart = 10
    rchunk = 10240
    assert npart * rchunk >= n_pad
    ept = e // NS
    cs = 2000
    g = 6144
    flush_at = 4096
    sub = 256
    nbank = 2
    nzc = rchunk // 512
    per = -(-nzc // NS)
    nwc = rchunk // sub
    perw = -(-nwc // NS)
    assert ept % cs == 0 and rchunk % 512 == 0 and rchunk % sub == 0

    @functools.partial(
        pl.kernel,
        out_type=jax.ShapeDtypeStruct((npart * rchunk, 64), jnp.float32),
        mesh=plsc.VectorSubcoreMesh(**_MESH),
        compiler_params=_SC_PARAMS,
        scratch_types=[
            pltpu.VMEM((cs,), jnp.int32),
            pltpu.VMEM((cs,), jnp.int32),
            pltpu.VMEM((g + sub * nbank,), jnp.int32),
            pltpu.VMEM((g + sub * nbank,), jnp.int32),
            pltpu.VMEM((sub, 64), jnp.float32),
            pltpu.VMEM((sub, 64), jnp.float32),
            pltpu.VMEM((512, 64), jnp.float32),
            pltpu.VMEM_SHARED((rchunk + 512, 64), jnp.float32),
            pltpu.SemaphoreType.DMA,
            pltpu.SemaphoreType.DMA,
            pltpu.SemaphoreType.DMA,
            pltpu.SemaphoreType.DMA,
        ],
    )
    def k(src_h, dst_h, hs_h, zr_h, out_h, srcb, dstb, csrc, cdst, rb0,
          rb1, zbuf_v, acc_sh, gs0, gs1, ss0, ss1):
        rows_b = [rb0, rb1]
        gsems = [gs0, gs1]
        ssems = [ss0, ss1]
        c = lax.axis_index("c")
        s = lax.axis_index("s")
        trash = jnp.int32(rchunk + s * 16)
        zsent = jnp.zeros((LN,), jnp.int32)
        tsent = jnp.full((LN,), 1, jnp.int32) * trash

        pltpu.sync_copy(zr_h, zbuf_v)

        def start_gather(r, k):
            pltpu.async_copy(
                hs_h.at[csrc.at[pl.ds(r * sub, sub)]], rows_b[k], gsems[k])

        def wait_gather(r, k):
            pltpu.make_async_copy(
                hs_h.at[csrc.at[pl.ds(r * sub, sub)]], rows_b[k],
                gsems[k]).wait()

        def start_scatter(r, k):
            pltpu.async_copy(
                rows_b[k], acc_sh.at[cdst.at[pl.ds(r * sub, sub)]],
                ssems[k], add=True)

        def wait_scatter(r, k):
            pltpu.make_async_copy(
                rows_b[k], acc_sh.at[cdst.at[pl.ds(r * sub, sub)]],
                ssems[k]).wait()

        def flush(cnt):
            for t in range(sub // LN):
                csrc[pl.ds(cnt + t * LN, LN)] = zsent
                cdst[pl.ds(cnt + t * LN, LN)] = tsent
            nr = (cnt + (sub - 1)) // sub
            nq = (nr + (nbank - 1)) // nbank

            def wave(q):
                for k in range(nbank):
                    r = q * nbank + k

                    @pl.when(q > 0)
                    def _():
                        wait_scatter(r - nbank, k)

                    @pl.when(r < nr)
                    def _():
                        start_gather(r, k)

                for k in range(nbank):
                    r = q * nbank + k

                    @pl.when(r < nr)
                    def _():
                        wait_gather(r, k)
                        start_scatter(r, k)

            pl.loop(0, nq)(wave)

            for k in range(nbank):
                r = (nq - 1) * nbank + k

                @pl.when(r < nr)
                def _():
                    wait_scatter(r, k)

            return jnp.int32(0)

        for p in range(npart // NC):
            chunk_id = c * (npart // NC) + p
            lo = chunk_id * rchunk
            hi = lo + rchunk

            @pl.loop(0, per)
            def _(j):
                ci = s * per + j

                @pl.when(ci < nzc)
                def _():
                    pltpu.sync_copy(zbuf_v, acc_sh.at[pl.ds(ci * 512, 512)])

            plsc.subcore_barrier()

            base0 = s * ept

            def chunk_body(j, cnt):
                base = base0 + j * cs
                pltpu.sync_copy(src_h.at[pl.ds(base, cs)], srcb)
                pltpu.sync_copy(dst_h.at[pl.ds(base, cs)], dstb)

                def vreg_body(t, cnt):
                    d = dstb[pl.ds(t * LN, LN)]
                    sv = srcb[pl.ds(t * LN, LN)]
                    m = (d >= lo) & (d < hi)
                    plsc.store_compressed(cdst.at[pl.ds(cnt, LN)], d - lo,
                                          mask=m)
                    plsc.store_compressed(csrc.at[pl.ds(cnt, LN)], sv,
                                          mask=m)
                    return cnt + jnp.sum(jnp.where(m, 1, 0)).astype(
                        jnp.int32)

                cnt = pl.loop(0, cs // LN, init_carry=cnt, unroll=4)(
                    vreg_body)
                return lax.cond(cnt >= flush_at, flush, lambda x: x, cnt)

            cnt = pl.loop(0, ept // cs, init_carry=jnp.int32(0))(chunk_body)
            lax.cond(cnt > 0, flush, lambda x: x, cnt)

            plsc.subcore_barrier()

            @pl.loop(0, perw)
            def _(j):
                ci = s * perw + j

                @pl.when(ci < nwc)
                def _():
                    pltpu.sync_copy(acc_sh.at[pl.ds(ci * sub, sub)],
                                    rows_b[0])
                    pltpu.sync_copy(
                        rows_b[0], out_h.at[pl.ds(lo + ci * sub, sub)])

            plsc.subcore_barrier()

    return k(src, dst, hs, zrows)


def _tc_prep(deg_parts, x_pad, n, n_pad):
    blk = 256
    grid = n_pad // blk

    def body(deg_ref, x_ref, dinv_ref, xs_ref, gs_ref):
        i = pl.program_id(0)
        deg = jnp.sum(deg_ref[...], axis=0) + 1.0
        dinv = lax.rsqrt(deg)
        dinv2 = dinv.reshape(blk, 1)
        dinv_ref[...] = dinv2
        xb = x_ref[...]
        xs_ref[...] = xb * dinv2
        m = (xb[:, 2:3] == 1.0).astype(jnp.float32)
        terms = jnp.concatenate(
            [xb[:, 0:2] * m, xb[:, 2:5], m, jnp.zeros((blk, 2),
                                                      jnp.float32)],
            axis=1)
        gsum = jnp.sum(terms, axis=0).reshape(1, 8)

        @pl.when(i == 0)
        def _():
            gs_ref[...] = jnp.zeros((1, 8), jnp.float32)

        gs_ref[...] += gsum

    return pl.pallas_call(
        body,
        grid=(grid,),
        in_specs=[
            pl.BlockSpec((NW, blk), lambda i: (0, i)),
            pl.BlockSpec((blk, 8), lambda i: (i, 0)),
        ],
        out_specs=[
            pl.BlockSpec((blk, 1), lambda i: (i, 0)),
            pl.BlockSpec((blk, 8), lambda i: (i, 0)),
            pl.BlockSpec((1, 8), lambda i: (0, 0)),
        ],
        out_shape=[
            jax.ShapeDtypeStruct((n_pad, 1), jnp.float32),
            jax.ShapeDtypeStruct((n_pad, 8), jnp.float32),
            jax.ShapeDtypeStruct((1, 8), jnp.float32),
        ],
    )(deg_parts, x_pad)


def _tc_layer1(agg8, xs8, dinv, w1p, b1r, n_pad):
    blk = 256
    grid = n_pad // blk

    def body(ag_ref, xs_ref, dinv_ref, w_ref, b_ref, hs_ref):
        sagg = ag_ref[0] + ag_ref[1] + xs_ref[...]
        h = jnp.dot(sagg, w_ref[...], preferred_element_type=jnp.float32)
        dinv2 = dinv_ref[...]
        h = jnp.maximum(h * dinv2 + b_ref[...], 0.0)
        hs_ref[...] = h * dinv2

    return pl.pallas_call(
        body,
        grid=(grid,),
        in_specs=[
            pl.BlockSpec((NC, blk, 8), lambda i: (0, i, 0)),
            pl.BlockSpec((blk, 8), lambda i: (i, 0)),
            pl.BlockSpec((blk, 1), lambda i: (i, 0)),
            pl.BlockSpec((8, 64), lambda i: (0, 0)),
            pl.BlockSpec((1, 64), lambda i: (0, 0)),
        ],
        out_specs=pl.BlockSpec((blk, 64), lambda i: (i, 0)),
        out_shape=jax.ShapeDtypeStruct((n_pad, 64), jnp.float32),
    )(agg8, xs8, dinv, w1p, b1r)


def _tc_final(agg64, hs, dinv, w2, b2r, gf8, wp1a, wp1b, bp1r, wp2, bp2r,
              n, n_pad):
    blk = 256
    grid = n_pad // blk

    def body(ag_ref, hs_ref, dinv_ref, w2_ref, b2_ref, gf_ref, wa_ref,
             wb_ref, b3_ref, wc_ref, b4_ref, out_ref, acc_ref):
        i = pl.program_id(0)
        sagg = ag_ref[...] + hs_ref[...]
        h = jnp.dot(sagg, w2_ref[...], preferred_element_type=jnp.float32)
        h = jnp.maximum(h * dinv_ref[...] + b2_ref[...], 0.0)
        rid = lax.broadcasted_iota(jnp.int32, (blk, 1), 0) + i * blk
        h = jnp.where(rid < n, h, 0.0)

        @pl.when(i == 0)
        def _():
            acc_ref[...] = jnp.zeros((1, 64), jnp.float32)

        acc_ref[...] += jnp.sum(h, axis=0).reshape(1, 64)

        @pl.when(i == grid - 1)
        def _():
            emb = acc_ref[...] * (1.0 / n)
            hid = jnp.dot(emb, wa_ref[...],
                          preferred_element_type=jnp.float32)
            hid += jnp.dot(gf_ref[...], wb_ref[...],
                           preferred_element_type=jnp.float32)
            hid = jnp.maximum(hid + b3_ref[...], 0.0)
            raw = jnp.dot(hid, wc_ref[...],
                          preferred_element_type=jnp.float32) + b4_ref[...]
            sig = 1.0 / (1.0 + jnp.exp(-raw))
            out_ref[...] = 2.0 + sig * 3.0

    return pl.pallas_call(
        body,
        grid=(grid,),
        in_specs=[
            pl.BlockSpec((blk, 64), lambda i: (i, 0)),
            pl.BlockSpec((blk, 64), lambda i: (i, 0)),
            pl.BlockSpec((blk, 1), lambda i: (i, 0)),
            pl.BlockSpec((64, 64), lambda i: (0, 0)),
            pl.BlockSpec((1, 64), lambda i: (0, 0)),
            pl.BlockSpec((1, 8), lambda i: (0, 0)),
            pl.BlockSpec((64, 32), lambda i: (0, 0)),
            pl.BlockSpec((8, 32), lambda i: (0, 0)),
            pl.BlockSpec((1, 32), lambda i: (0, 0)),
            pl.BlockSpec((32, 2), lambda i: (0, 0)),
            pl.BlockSpec((1, 2), lambda i: (0, 0)),
        ],
        out_specs=pl.BlockSpec((1, 2), lambda i: (0, 0)),
        out_shape=jax.ShapeDtypeStruct((1, 2), jnp.float32),
        scratch_shapes=[pltpu.VMEM((1, 64), jnp.float32)],
    )(agg64, hs, dinv, w2, b2r, gf8, wp1a, wp1b, bp1r, wp2, bp2r)


def kernel(x, edge_index, W1, b1, W2, b2, Wp1, bp1, Wp2, bp2):
    n, f = x.shape
    e = edge_index.shape[1]
    n_pad = -(-n // 512) * 512
    h = W1.shape[1]
    assert h == 64 and f == 5

    e32 = edge_index.astype(jnp.int32)
    src, dst = e32[0], e32[1]

    x_pad = jnp.zeros((n_pad, 8), jnp.float32).at[:n, :f].set(x)
    w1p = jnp.zeros((8, h), jnp.float32).at[:f].set(W1)
    b1r = b1.reshape(1, h)
    b2r = b2.reshape(1, h)
    wp1a = Wp1[:h]
    wp1b = jnp.zeros((8, 32), jnp.float32).at[:6].set(Wp1[h:])
    bp1r = bp1.reshape(1, 32)
    bp2r = bp2.reshape(1, 2)
    z8 = jnp.zeros((512, 8), jnp.float32)
    zrows = jnp.zeros((512, 64), jnp.float32)

    deg_parts = _sc_deg(dst, n_pad, e)
    dinv, xs8, gsums = _tc_prep(deg_parts, x_pad, n, n_pad)

    s0m, s1m, s2, s3, s4, cntm = (gsums[0, j] for j in range(6))
    avg_l = jnp.where(cntm > 0, s0m / jnp.maximum(cntm, 1.0), 0.0)
    avg_m = jnp.where(cntm > 0, s1m / jnp.maximum(cntm, 1.0), 0.0)
    gf8 = jnp.stack([s2, s3, s4, s3 + s4, avg_l, avg_m,
                     jnp.float32(0), jnp.float32(0)]).reshape(1, 8)

    agg8 = _sc_agg8(src, dst, xs8, z8, n_pad, e)
    hs = _tc_layer1(agg8, xs8, dinv, w1p, b1r, n_pad)
    agg64 = _sc_agg64(src, dst, hs, zrows, n_pad, e)
    return _tc_final(agg64, hs, dinv, W2, b2r, gf8, wp1a, wp1b, bp1r, Wp2,
                     bp2r, n, n_pad)


# speedup vs baseline: 22.5747x; 1.0278x over previous
import functools

import jax
import jax.numpy as jnp
from jax import lax
from jax.experimental import pallas as pl
from jax.experimental.pallas import tpu as pltpu
from jax.experimental.pallas import tpu_sc as plsc

NC = 2
NS = 16
NW = NC * NS
LN = 16

_SC_PARAMS = pltpu.CompilerParams(
    needs_layout_passes=False, use_tc_tiling_on_sc=False)

_MESH = dict(core_axis_name="c", subcore_axis_name="s")


def _sc_deg(dst, n_pad, e):
    ew = e // NW
    cd = 2000
    assert ew % cd == 0 and n_pad % LN == 0

    @functools.partial(
        pl.kernel,
        out_type=jax.ShapeDtypeStruct((NW, n_pad), jnp.float32),
        mesh=plsc.VectorSubcoreMesh(**_MESH),
        compiler_params=_SC_PARAMS,
        scratch_types=[
            pltpu.VMEM((n_pad,), jnp.float32),
            pltpu.VMEM((cd,), jnp.int32),
        ],
    )
    def k(dst_h, out_h, deg_v, idx_v):
        c = lax.axis_index("c")
        s = lax.axis_index("s")
        w = s * NC + c
        z = jnp.zeros((LN,), jnp.float32)

        @pl.loop(0, n_pad // LN, unroll=8)
        def _(i):
            deg_v[pl.ds(i * LN, LN)] = z

        ones = jnp.ones((LN,), jnp.float32)
        base0 = w * ew

        @pl.loop(0, ew // cd)
        def _(j):
            pltpu.sync_copy(dst_h.at[pl.ds(base0 + j * cd, cd)], idx_v)

            @pl.loop(0, cd // LN, unroll=8)
            def _(t):
                idx = idx_v[pl.ds(t * LN, LN)]
                plsc.addupdate_scatter(deg_v, [idx], ones)

        pltpu.sync_copy(deg_v, out_h.at[w])

    return k(dst)


def _sc_agg8(src, dst, xs8, z8, n_pad, e):
    ew = e // NW
    ce = 2000
    zc = 512
    nch = n_pad // zc
    per = -(-nch // NS)
    assert ew % ce == 0 and n_pad % zc == 0

    @functools.partial(
        pl.kernel,
        out_type=jax.ShapeDtypeStruct((NC, n_pad, 8), jnp.float32),
        mesh=plsc.VectorSubcoreMesh(**_MESH),
        compiler_params=_SC_PARAMS,
        scratch_types=[
            pltpu.VMEM((ce,), jnp.int32),
            pltpu.VMEM((ce,), jnp.int32),
            pltpu.VMEM((ce, 8), jnp.float32),
            pltpu.VMEM((zc, 8), jnp.float32),
            pltpu.VMEM_SHARED((n_pad, 8), jnp.float32),
            pltpu.SemaphoreType.DMA,
        ],
    )
    def k(src_h, dst_h, xs_h, z8_h, out_h, src_v, dst_v, rows_v, buf_v,
          acc_sh, sem):
        c = lax.axis_index("c")
        s = lax.axis_index("s")
        w = s * NC + c

        pltpu.sync_copy(z8_h, buf_v)

        @pl.loop(0, per)
        def _(j):
            ci = s * per + j

            @pl.when(ci < nch)
            def _():
                pltpu.sync_copy(buf_v, acc_sh.at[pl.ds(ci * zc, zc)])

        plsc.subcore_barrier()

        base0 = w * ew

        @pl.loop(0, ew // ce)
        def _(j):
            base = base0 + j * ce
            pltpu.sync_copy(src_h.at[pl.ds(base, ce)], src_v)
            pltpu.sync_copy(dst_h.at[pl.ds(base, ce)], dst_v)
            pltpu.async_copy(xs_h.at[src_v], rows_v, sem).wait()
            pltpu.sync_copy(rows_v, acc_sh.at[dst_v], add=True)

        plsc.subcore_barrier()

        @pl.loop(0, per)
        def _(j):
            ci = s * per + j

            @pl.when(ci < nch)
            def _():
                pltpu.sync_copy(acc_sh.at[pl.ds(ci * zc, zc)], buf_v)
                pltpu.sync_copy(buf_v, out_h.at[c, pl.ds(ci * zc, zc)])

    return k(src, dst, xs8, z8)


def _sc_agg64(src, dst, hs, zrows, n_pad, e):
    npart = 10
    rchunk = 10240
    assert npart * rchunk >= n_pad
    ept = e // NS
    cs = 2000
    g = 6144
    flush_at = 4096
    sub = 256
    nbank = 4
    nzc = rchunk // sub
    per = -(-nzc // NS)
    nwc = rchunk // sub
    perw = -(-nwc // NS)
    assert ept % cs == 0 and rchunk % sub == 0

    @functools.partial(
        pl.kernel,
        out_type=jax.ShapeDtypeStruct((npart * rchunk, 64), jnp.float32),
        mesh=plsc.VectorSubcoreMesh(**_MESH),
        compiler_params=_SC_PARAMS,
        scratch_types=[
            pltpu.VMEM((cs,), jnp.int32),
            pltpu.VMEM((cs,), jnp.int32),
            pltpu.VMEM((g + sub * nbank,), jnp.int32),
            pltpu.VMEM((g + sub * nbank,), jnp.int32),
            pltpu.VMEM((sub, 64), jnp.float32),
            pltpu.VMEM((sub, 64), jnp.float32),
            pltpu.VMEM((sub, 64), jnp.float32),
            pltpu.VMEM((sub, 64), jnp.float32),
            pltpu.VMEM_SHARED((rchunk + 512, 64), jnp.float32),
            pltpu.SemaphoreType.DMA,
            pltpu.SemaphoreType.DMA,
            pltpu.SemaphoreType.DMA,
            pltpu.SemaphoreType.DMA,
            pltpu.SemaphoreType.DMA,
            pltpu.SemaphoreType.DMA,
            pltpu.SemaphoreType.DMA,
            pltpu.SemaphoreType.DMA,
        ],
    )
    def k(src_h, dst_h, hs_h, zr_h, out_h, srcb, dstb, csrc, cdst, rb0,
          rb1, rb2, rb3, acc_sh, gs0, gs1, gs2, gs3, ss0, ss1, ss2, ss3):
        rows_b = [rb0, rb1, rb2, rb3]
        gsems = [gs0, gs1, gs2, gs3]
        ssems = [ss0, ss1, ss2, ss3]
        c = lax.axis_index("c")
        s = lax.axis_index("s")
        trash = jnp.int32(rchunk + s * 16)
        zsent = jnp.zeros((LN,), jnp.int32)
        tsent = jnp.full((LN,), 1, jnp.int32) * trash


        def start_gather(r, k):
            pltpu.async_copy(
                hs_h.at[csrc.at[pl.ds(r * sub, sub)]], rows_b[k], gsems[k])

        def wait_gather(r, k):
            pltpu.make_async_copy(
                hs_h.at[csrc.at[pl.ds(r * sub, sub)]], rows_b[k],
                gsems[k]).wait()

        def start_scatter(r, k):
            pltpu.async_copy(
                rows_b[k], acc_sh.at[cdst.at[pl.ds(r * sub, sub)]],
                ssems[k], add=True)

        def wait_scatter(r, k):
            pltpu.make_async_copy(
                rows_b[k], acc_sh.at[cdst.at[pl.ds(r * sub, sub)]],
                ssems[k]).wait()

        def flush(cnt):
            for t in range(sub // LN):
                csrc[pl.ds(cnt + t * LN, LN)] = zsent
                cdst[pl.ds(cnt + t * LN, LN)] = tsent
            nr = (cnt + (sub - 1)) // sub
            nq = (nr + (nbank - 1)) // nbank

            def wave(q):
                for k in range(nbank):
                    r = q * nbank + k

                    @pl.when(q > 0)
                    def _():
                        wait_scatter(r - nbank, k)

                    @pl.when(r < nr)
                    def _():
                        start_gather(r, k)

                for k in range(nbank):
                    r = q * nbank + k

                    @pl.when(r < nr)
                    def _():
                        wait_gather(r, k)
                        start_scatter(r, k)

            pl.loop(0, nq)(wave)

            for k in range(nbank):
                r = (nq - 1) * nbank + k

                @pl.when(r < nr)
                def _():
                    wait_scatter(r, k)

            return jnp.int32(0)

        for p in range(npart // NC):
            chunk_id = c * (npart // NC) + p
            lo = chunk_id * rchunk
            hi = lo + rchunk

            pltpu.sync_copy(zr_h, rb0)

            @pl.loop(0, per)
            def _(j):
                ci = s * per + j

                @pl.when(ci < nzc)
                def _():
                    pltpu.sync_copy(rb0, acc_sh.at[pl.ds(ci * sub, sub)])

            plsc.subcore_barrier()

            base0 = s * ept

            def chunk_body(j, cnt):
                base = base0 + j * cs
                pltpu.sync_copy(src_h.at[pl.ds(base, cs)], srcb)
                pltpu.sync_copy(dst_h.at[pl.ds(base, cs)], dstb)

                def vreg_body(t, cnt):
                    d = dstb[pl.ds(t * LN, LN)]
                    sv = srcb[pl.ds(t * LN, LN)]
                    m = (d >= lo) & (d < hi)
                    plsc.store_compressed(cdst.at[pl.ds(cnt, LN)], d - lo,
                                          mask=m)
                    plsc.store_compressed(csrc.at[pl.ds(cnt, LN)], sv,
                                          mask=m)
                    return cnt + jnp.sum(jnp.where(m, 1, 0)).astype(
                        jnp.int32)

                cnt = pl.loop(0, cs // LN, init_carry=cnt, unroll=4)(
                    vreg_body)
                return lax.cond(cnt >= flush_at, flush, lambda x: x, cnt)

            cnt = pl.loop(0, ept // cs, init_carry=jnp.int32(0))(chunk_body)
            lax.cond(cnt > 0, flush, lambda x: x, cnt)

            plsc.subcore_barrier()

            @pl.loop(0, perw)
            def _(j):
                ci = s * perw + j

                @pl.when(ci < nwc)
                def _():
                    pltpu.sync_copy(acc_sh.at[pl.ds(ci * sub, sub)],
                                    rows_b[0])
                    pltpu.sync_copy(
                        rows_b[0], out_h.at[pl.ds(lo + ci * sub, sub)])

            plsc.subcore_barrier()

    return k(src, dst, hs, zrows)


def _tc_prep(deg_parts, x_pad, n, n_pad):
    blk = 256
    grid = n_pad // blk

    def body(deg_ref, x_ref, dinv_ref, xs_ref, gs_ref):
        i = pl.program_id(0)
        deg = jnp.sum(deg_ref[...], axis=0) + 1.0
        dinv = lax.rsqrt(deg)
        dinv2 = dinv.reshape(blk, 1)
        dinv_ref[...] = dinv2
        xb = x_ref[...]
        xs_ref[...] = xb * dinv2
        m = (xb[:, 2:3] == 1.0).astype(jnp.float32)
        terms = jnp.concatenate(
            [xb[:, 0:2] * m, xb[:, 2:5], m, jnp.zeros((blk, 2),
                                                      jnp.float32)],
            axis=1)
        gsum = jnp.sum(terms, axis=0).reshape(1, 8)

        @pl.when(i == 0)
        def _():
            gs_ref[...] = jnp.zeros((1, 8), jnp.float32)

        gs_ref[...] += gsum

    return pl.pallas_call(
        body,
        grid=(grid,),
        in_specs=[
            pl.BlockSpec((NW, blk), lambda i: (0, i)),
            pl.BlockSpec((blk, 8), lambda i: (i, 0)),
        ],
        out_specs=[
            pl.BlockSpec((blk, 1), lambda i: (i, 0)),
            pl.BlockSpec((blk, 8), lambda i: (i, 0)),
            pl.BlockSpec((1, 8), lambda i: (0, 0)),
        ],
        out_shape=[
            jax.ShapeDtypeStruct((n_pad, 1), jnp.float32),
            jax.ShapeDtypeStruct((n_pad, 8), jnp.float32),
            jax.ShapeDtypeStruct((1, 8), jnp.float32),
        ],
    )(deg_parts, x_pad)


def _tc_layer1(agg8, xs8, dinv, w1p, b1r, n_pad):
    blk = 256
    grid = n_pad // blk

    def body(ag_ref, xs_ref, dinv_ref, w_ref, b_ref, hs_ref):
        sagg = ag_ref[0] + ag_ref[1] + xs_ref[...]
        h = jnp.dot(sagg, w_ref[...], preferred_element_type=jnp.float32)
        dinv2 = dinv_ref[...]
        h = jnp.maximum(h * dinv2 + b_ref[...], 0.0)
        hs_ref[...] = h * dinv2

    return pl.pallas_call(
        body,
        grid=(grid,),
        in_specs=[
            pl.BlockSpec((NC, blk, 8), lambda i: (0, i, 0)),
            pl.BlockSpec((blk, 8), lambda i: (i, 0)),
            pl.BlockSpec((blk, 1), lambda i: (i, 0)),
            pl.BlockSpec((8, 64), lambda i: (0, 0)),
            pl.BlockSpec((1, 64), lambda i: (0, 0)),
        ],
        out_specs=pl.BlockSpec((blk, 64), lambda i: (i, 0)),
        out_shape=jax.ShapeDtypeStruct((n_pad, 64), jnp.float32),
    )(agg8, xs8, dinv, w1p, b1r)


def _tc_final(agg64, hs, dinv, w2, b2r, gf8, wp1a, wp1b, bp1r, wp2, bp2r,
              n, n_pad):
    blk = 256
    grid = n_pad // blk

    def body(ag_ref, hs_ref, dinv_ref, w2_ref, b2_ref, gf_ref, wa_ref,
             wb_ref, b3_ref, wc_ref, b4_ref, out_ref, acc_ref):
        i = pl.program_id(0)
        sagg = ag_ref[...] + hs_ref[...]
        h = jnp.dot(sagg, w2_ref[...], preferred_element_type=jnp.float32)
        h = jnp.maximum(h * dinv_ref[...] + b2_ref[...], 0.0)
        rid = lax.broadcasted_iota(jnp.int32, (blk, 1), 0) + i * blk
        h = jnp.where(rid < n, h, 0.0)

        @pl.when(i == 0)
        def _():
            acc_ref[...] = jnp.zeros((1, 64), jnp.float32)

        acc_ref[...] += jnp.sum(h, axis=0).reshape(1, 64)

        @pl.when(i == grid - 1)
        def _():
            emb = acc_ref[...] * (1.0 / n)
            hid = jnp.dot(emb, wa_ref[...],
                          preferred_element_type=jnp.float32)
            hid += jnp.dot(gf_ref[...], wb_ref[...],
                           preferred_element_type=jnp.float32)
            hid = jnp.maximum(hid + b3_ref[...], 0.0)
            raw = jnp.dot(hid, wc_ref[...],
                          preferred_element_type=jnp.float32) + b4_ref[...]
            sig = 1.0 / (1.0 + jnp.exp(-raw))
            out_ref[...] = 2.0 + sig * 3.0

    return pl.pallas_call(
        body,
        grid=(grid,),
        in_specs=[
            pl.BlockSpec((blk, 64), lambda i: (i, 0)),
            pl.BlockSpec((blk, 64), lambda i: (i, 0)),
            pl.BlockSpec((blk, 1), lambda i: (i, 0)),
            pl.BlockSpec((64, 64), lambda i: (0, 0)),
            pl.BlockSpec((1, 64), lambda i: (0, 0)),
            pl.BlockSpec((1, 8), lambda i: (0, 0)),
            pl.BlockSpec((64, 32), lambda i: (0, 0)),
            pl.BlockSpec((8, 32), lambda i: (0, 0)),
            pl.BlockSpec((1, 32), lambda i: (0, 0)),
            pl.BlockSpec((32, 2), lambda i: (0, 0)),
            pl.BlockSpec((1, 2), lambda i: (0, 0)),
        ],
        out_specs=pl.BlockSpec((1, 2), lambda i: (0, 0)),
        out_shape=jax.ShapeDtypeStruct((1, 2), jnp.float32),
        scratch_shapes=[pltpu.VMEM((1, 64), jnp.float32)],
    )(agg64, hs, dinv, w2, b2r, gf8, wp1a, wp1b, bp1r, wp2, bp2r)


def kernel(x, edge_index, W1, b1, W2, b2, Wp1, bp1, Wp2, bp2):
    n, f = x.shape
    e = edge_index.shape[1]
    n_pad = -(-n // 512) * 512
    h = W1.shape[1]
    assert h == 64 and f == 5

    e32 = edge_index.astype(jnp.int32)
    src, dst = e32[0], e32[1]

    x_pad = jnp.zeros((n_pad, 8), jnp.float32).at[:n, :f].set(x)
    w1p = jnp.zeros((8, h), jnp.float32).at[:f].set(W1)
    b1r = b1.reshape(1, h)
    b2r = b2.reshape(1, h)
    wp1a = Wp1[:h]
    wp1b = jnp.zeros((8, 32), jnp.float32).at[:6].set(Wp1[h:])
    bp1r = bp1.reshape(1, 32)
    bp2r = bp2.reshape(1, 2)
    z8 = jnp.zeros((512, 8), jnp.float32)
    zrows = jnp.zeros((256, 64), jnp.float32)

    deg_parts = _sc_deg(dst, n_pad, e)
    dinv, xs8, gsums = _tc_prep(deg_parts, x_pad, n, n_pad)

    s0m, s1m, s2, s3, s4, cntm = (gsums[0, j] for j in range(6))
    avg_l = jnp.where(cntm > 0, s0m / jnp.maximum(cntm, 1.0), 0.0)
    avg_m = jnp.where(cntm > 0, s1m / jnp.maximum(cntm, 1.0), 0.0)
    gf8 = jnp.stack([s2, s3, s4, s3 + s4, avg_l, avg_m,
                     jnp.float32(0), jnp.float32(0)]).reshape(1, 8)

    agg8 = _sc_agg8(src, dst, xs8, z8, n_pad, e)
    hs = _tc_layer1(agg8, xs8, dinv, w1p, b1r, n_pad)
    agg64 = _sc_agg64(src, dst, hs, zrows, n_pad, e)
    return _tc_final(agg64, hs, dinv, W2, b2r, gf8, wp1a, wp1b, bp1r, Wp2,
                     bp2r, n, n_pad)


# speedup vs baseline: 34.8639x; 1.5444x over previous
import functools

import jax
import jax.numpy as jnp
from jax import lax
from jax.experimental import pallas as pl
from jax.experimental.pallas import tpu as pltpu
from jax.experimental.pallas import tpu_sc as plsc

NC = 2
NS = 16
NW = NC * NS
LN = 16

_SC_PARAMS = pltpu.CompilerParams(
    needs_layout_passes=False, use_tc_tiling_on_sc=False)

_MESH = dict(core_axis_name="c", subcore_axis_name="s")


def _sc_deg(dst, n_pad, e):
    ew = e // NW
    cd = 2000
    assert ew % cd == 0 and n_pad % LN == 0

    @functools.partial(
        pl.kernel,
        out_type=jax.ShapeDtypeStruct((NW, n_pad), jnp.float32),
        mesh=plsc.VectorSubcoreMesh(**_MESH),
        compiler_params=_SC_PARAMS,
        scratch_types=[
            pltpu.VMEM((n_pad,), jnp.float32),
            pltpu.VMEM((cd,), jnp.int32),
        ],
    )
    def k(dst_h, out_h, deg_v, idx_v):
        c = lax.axis_index("c")
        s = lax.axis_index("s")
        w = s * NC + c
        z = jnp.zeros((LN,), jnp.float32)

        @pl.loop(0, n_pad // LN, unroll=8)
        def _(i):
            deg_v[pl.ds(i * LN, LN)] = z

        ones = jnp.ones((LN,), jnp.float32)
        base0 = w * ew

        @pl.loop(0, ew // cd)
        def _(j):
            pltpu.sync_copy(dst_h.at[pl.ds(base0 + j * cd, cd)], idx_v)

            @pl.loop(0, cd // LN, unroll=8)
            def _(t):
                idx = idx_v[pl.ds(t * LN, LN)]
                plsc.addupdate_scatter(deg_v, [idx], ones)

        pltpu.sync_copy(deg_v, out_h.at[w])

    return k(dst)


def _sc_agg8(src, dst, xs8, z8, n_pad, e):
    ew = e // NW
    ce = 2000
    zc = 512
    nch = n_pad // zc
    per = -(-nch // NS)
    assert ew % ce == 0 and n_pad % zc == 0

    @functools.partial(
        pl.kernel,
        out_type=jax.ShapeDtypeStruct((NC, n_pad, 8), jnp.float32),
        mesh=plsc.VectorSubcoreMesh(**_MESH),
        compiler_params=_SC_PARAMS,
        scratch_types=[
            pltpu.VMEM((ce,), jnp.int32),
            pltpu.VMEM((ce,), jnp.int32),
            pltpu.VMEM((ce, 8), jnp.float32),
            pltpu.VMEM((zc, 8), jnp.float32),
            pltpu.VMEM_SHARED((n_pad, 8), jnp.float32),
            pltpu.SemaphoreType.DMA,
        ],
    )
    def k(src_h, dst_h, xs_h, z8_h, out_h, src_v, dst_v, rows_v, buf_v,
          acc_sh, sem):
        c = lax.axis_index("c")
        s = lax.axis_index("s")
        w = s * NC + c

        pltpu.sync_copy(z8_h, buf_v)

        @pl.loop(0, per)
        def _(j):
            ci = s * per + j

            @pl.when(ci < nch)
            def _():
                pltpu.sync_copy(buf_v, acc_sh.at[pl.ds(ci * zc, zc)])

        plsc.subcore_barrier()

        base0 = w * ew

        @pl.loop(0, ew // ce)
        def _(j):
            base = base0 + j * ce
            pltpu.sync_copy(src_h.at[pl.ds(base, ce)], src_v)
            pltpu.sync_copy(dst_h.at[pl.ds(base, ce)], dst_v)
            pltpu.async_copy(xs_h.at[src_v], rows_v, sem).wait()
            pltpu.sync_copy(rows_v, acc_sh.at[dst_v], add=True)

        plsc.subcore_barrier()

        @pl.loop(0, per)
        def _(j):
            ci = s * per + j

            @pl.when(ci < nch)
            def _():
                pltpu.sync_copy(acc_sh.at[pl.ds(ci * zc, zc)], buf_v)
                pltpu.sync_copy(buf_v, out_h.at[c, pl.ds(ci * zc, zc)])

    return k(src, dst, xs8, z8)


def _sc_agg64(src, dst, hs, zrows, n_pad, e):
    npart = 6
    rchunk = 17408
    assert npart * rchunk >= n_pad
    ept = e // NS
    cs = 2000
    g = 6144
    flush_at = 4096
    sub = 256
    nbank = 4
    nzc = rchunk // sub
    per = -(-nzc // NS)
    nwc = rchunk // sub
    perw = -(-nwc // NS)
    assert ept % cs == 0 and rchunk % sub == 0

    @functools.partial(
        pl.kernel,
        out_type=jax.ShapeDtypeStruct((npart * rchunk, 64), jnp.bfloat16),
        mesh=plsc.VectorSubcoreMesh(**_MESH),
        compiler_params=_SC_PARAMS,
        scratch_types=[
            pltpu.VMEM((cs,), jnp.int32),
            pltpu.VMEM((cs,), jnp.int32),
            pltpu.VMEM((g + sub * nbank,), jnp.int32),
            pltpu.VMEM((g + sub * nbank,), jnp.int32),
            pltpu.VMEM((sub, 64), jnp.bfloat16),
            pltpu.VMEM((sub, 64), jnp.bfloat16),
            pltpu.VMEM((sub, 64), jnp.bfloat16),
            pltpu.VMEM((sub, 64), jnp.bfloat16),
            pltpu.VMEM_SHARED((rchunk + 512, 64), jnp.bfloat16),
            pltpu.SemaphoreType.DMA,
            pltpu.SemaphoreType.DMA,
            pltpu.SemaphoreType.DMA,
            pltpu.SemaphoreType.DMA,
            pltpu.SemaphoreType.DMA,
            pltpu.SemaphoreType.DMA,
            pltpu.SemaphoreType.DMA,
            pltpu.SemaphoreType.DMA,
        ],
    )
    def k(src_h, dst_h, hs_h, zr_h, out_h, srcb, dstb, csrc, cdst, rb0,
          rb1, rb2, rb3, acc_sh, gs0, gs1, gs2, gs3, ss0, ss1, ss2, ss3):
        rows_b = [rb0, rb1, rb2, rb3]
        gsems = [gs0, gs1, gs2, gs3]
        ssems = [ss0, ss1, ss2, ss3]
        c = lax.axis_index("c")
        s = lax.axis_index("s")
        trash = jnp.int32(rchunk + s * 16)
        zsent = jnp.zeros((LN,), jnp.int32)
        tsent = jnp.full((LN,), 1, jnp.int32) * trash


        def start_gather(r, k):
            pltpu.async_copy(
                hs_h.at[csrc.at[pl.ds(r * sub, sub)]], rows_b[k], gsems[k])

        def wait_gather(r, k):
            pltpu.make_async_copy(
                hs_h.at[csrc.at[pl.ds(r * sub, sub)]], rows_b[k],
                gsems[k]).wait()

        def start_scatter(r, k):
            pltpu.async_copy(
                rows_b[k], acc_sh.at[cdst.at[pl.ds(r * sub, sub)]],
                ssems[k], add=True)

        def wait_scatter(r, k):
            pltpu.make_async_copy(
                rows_b[k], acc_sh.at[cdst.at[pl.ds(r * sub, sub)]],
                ssems[k]).wait()

        def flush(cnt):
            for t in range(sub // LN):
                csrc[pl.ds(cnt + t * LN, LN)] = zsent
                cdst[pl.ds(cnt + t * LN, LN)] = tsent
            nr = (cnt + (sub - 1)) // sub
            nq = (nr + (nbank - 1)) // nbank

            def wave(q):
                for k in range(nbank):
                    r = q * nbank + k

                    @pl.when(q > 0)
                    def _():
                        wait_scatter(r - nbank, k)

                    @pl.when(r < nr)
                    def _():
                        start_gather(r, k)

                for k in range(nbank):
                    r = q * nbank + k

                    @pl.when(r < nr)
                    def _():
                        wait_gather(r, k)
                        start_scatter(r, k)

            pl.loop(0, nq)(wave)

            for k in range(nbank):
                r = (nq - 1) * nbank + k

                @pl.when(r < nr)
                def _():
                    wait_scatter(r, k)

            return jnp.int32(0)

        for p in range(npart // NC):
            chunk_id = c * (npart // NC) + p
            lo = chunk_id * rchunk
            hi = lo + rchunk

            pltpu.sync_copy(zr_h, rb0)

            @pl.loop(0, per)
            def _(j):
                ci = s * per + j

                @pl.when(ci < nzc)
                def _():
                    pltpu.sync_copy(rb0, acc_sh.at[pl.ds(ci * sub, sub)])

            plsc.subcore_barrier()

            base0 = s * ept

            def chunk_body(j, cnt):
                base = base0 + j * cs
                pltpu.sync_copy(src_h.at[pl.ds(base, cs)], srcb)
                pltpu.sync_copy(dst_h.at[pl.ds(base, cs)], dstb)

                def vreg_body(t, cnt):
                    d = dstb[pl.ds(t * LN, LN)]
                    sv = srcb[pl.ds(t * LN, LN)]
                    m = (d >= lo) & (d < hi)
                    plsc.store_compressed(cdst.at[pl.ds(cnt, LN)], d - lo,
                                          mask=m)
                    plsc.store_compressed(csrc.at[pl.ds(cnt, LN)], sv,
                                          mask=m)
                    return cnt + jnp.sum(jnp.where(m, 1, 0)).astype(
                        jnp.int32)

                cnt = pl.loop(0, cs // LN, init_carry=cnt, unroll=4)(
                    vreg_body)
                return lax.cond(cnt >= flush_at, flush, lambda x: x, cnt)

            cnt = pl.loop(0, ept // cs, init_carry=jnp.int32(0))(chunk_body)
            lax.cond(cnt > 0, flush, lambda x: x, cnt)

            plsc.subcore_barrier()

            @pl.loop(0, perw)
            def _(j):
                ci = s * perw + j

                @pl.when(ci < nwc)
                def _():
                    pltpu.sync_copy(acc_sh.at[pl.ds(ci * sub, sub)],
                                    rows_b[0])
                    pltpu.sync_copy(
                        rows_b[0], out_h.at[pl.ds(lo + ci * sub, sub)])

            plsc.subcore_barrier()

    return k(src, dst, hs, zrows)


def _tc_prep(deg_parts, x_pad, n, n_pad):
    blk = 256
    grid = n_pad // blk

    def body(deg_ref, x_ref, dinv_ref, xs_ref, gs_ref):
        i = pl.program_id(0)
        deg = jnp.sum(deg_ref[...], axis=0) + 1.0
        dinv = lax.rsqrt(deg)
        dinv2 = dinv.reshape(blk, 1)
        dinv_ref[...] = dinv2
        xb = x_ref[...]
        xs_ref[...] = xb * dinv2
        m = (xb[:, 2:3] == 1.0).astype(jnp.float32)
        terms = jnp.concatenate(
            [xb[:, 0:2] * m, xb[:, 2:5], m, jnp.zeros((blk, 2),
                                                      jnp.float32)],
            axis=1)
        gsum = jnp.sum(terms, axis=0).reshape(1, 8)

        @pl.when(i == 0)
        def _():
            gs_ref[...] = jnp.zeros((1, 8), jnp.float32)

        gs_ref[...] += gsum

    return pl.pallas_call(
        body,
        grid=(grid,),
        in_specs=[
            pl.BlockSpec((NW, blk), lambda i: (0, i)),
            pl.BlockSpec((blk, 8), lambda i: (i, 0)),
        ],
        out_specs=[
            pl.BlockSpec((blk, 1), lambda i: (i, 0)),
            pl.BlockSpec((blk, 8), lambda i: (i, 0)),
            pl.BlockSpec((1, 8), lambda i: (0, 0)),
        ],
        out_shape=[
            jax.ShapeDtypeStruct((n_pad, 1), jnp.float32),
            jax.ShapeDtypeStruct((n_pad, 8), jnp.float32),
            jax.ShapeDtypeStruct((1, 8), jnp.float32),
        ],
    )(deg_parts, x_pad)


def _tc_layer1(agg8, xs8, dinv, w1p, b1r, n_pad):
    blk = 256
    grid = n_pad // blk

    def body(ag_ref, xs_ref, dinv_ref, w_ref, b_ref, hs_ref):
        sagg = ag_ref[0] + ag_ref[1] + xs_ref[...]
        h = jnp.dot(sagg, w_ref[...], preferred_element_type=jnp.float32)
        dinv2 = dinv_ref[...]
        h = jnp.maximum(h * dinv2 + b_ref[...], 0.0)
        hs_ref[...] = (h * dinv2).astype(jnp.bfloat16)

    return pl.pallas_call(
        body,
        grid=(grid,),
        in_specs=[
            pl.BlockSpec((NC, blk, 8), lambda i: (0, i, 0)),
            pl.BlockSpec((blk, 8), lambda i: (i, 0)),
            pl.BlockSpec((blk, 1), lambda i: (i, 0)),
            pl.BlockSpec((8, 64), lambda i: (0, 0)),
            pl.BlockSpec((1, 64), lambda i: (0, 0)),
        ],
        out_specs=pl.BlockSpec((blk, 64), lambda i: (i, 0)),
        out_shape=jax.ShapeDtypeStruct((n_pad, 64), jnp.bfloat16),
    )(agg8, xs8, dinv, w1p, b1r)


def _tc_final(agg64, hs, dinv, w2, b2r, gf8, wp1a, wp1b, bp1r, wp2, bp2r,
              n, n_pad):
    blk = 256
    grid = n_pad // blk

    def body(ag_ref, hs_ref, dinv_ref, w2_ref, b2_ref, gf_ref, wa_ref,
             wb_ref, b3_ref, wc_ref, b4_ref, out_ref, acc_ref):
        i = pl.program_id(0)
        sagg = (ag_ref[...].astype(jnp.float32)
                + hs_ref[...].astype(jnp.float32))
        h = jnp.dot(sagg, w2_ref[...], preferred_element_type=jnp.float32)
        h = jnp.maximum(h * dinv_ref[...] + b2_ref[...], 0.0)
        rid = lax.broadcasted_iota(jnp.int32, (blk, 1), 0) + i * blk
        h = jnp.where(rid < n, h, 0.0)

        @pl.when(i == 0)
        def _():
            acc_ref[...] = jnp.zeros((1, 64), jnp.float32)

        acc_ref[...] += jnp.sum(h, axis=0).reshape(1, 64)

        @pl.when(i == grid - 1)
        def _():
            emb = acc_ref[...] * (1.0 / n)
            hid = jnp.dot(emb, wa_ref[...],
                          preferred_element_type=jnp.float32)
            hid += jnp.dot(gf_ref[...], wb_ref[...],
                           preferred_element_type=jnp.float32)
            hid = jnp.maximum(hid + b3_ref[...], 0.0)
            raw = jnp.dot(hid, wc_ref[...],
                          preferred_element_type=jnp.float32) + b4_ref[...]
            sig = 1.0 / (1.0 + jnp.exp(-raw))
            out_ref[...] = 2.0 + sig * 3.0

    return pl.pallas_call(
        body,
        grid=(grid,),
        in_specs=[
            pl.BlockSpec((blk, 64), lambda i: (i, 0)),
            pl.BlockSpec((blk, 64), lambda i: (i, 0)),
            pl.BlockSpec((blk, 1), lambda i: (i, 0)),
            pl.BlockSpec((64, 64), lambda i: (0, 0)),
            pl.BlockSpec((1, 64), lambda i: (0, 0)),
            pl.BlockSpec((1, 8), lambda i: (0, 0)),
            pl.BlockSpec((64, 32), lambda i: (0, 0)),
            pl.BlockSpec((8, 32), lambda i: (0, 0)),
            pl.BlockSpec((1, 32), lambda i: (0, 0)),
            pl.BlockSpec((32, 2), lambda i: (0, 0)),
            pl.BlockSpec((1, 2), lambda i: (0, 0)),
        ],
        out_specs=pl.BlockSpec((1, 2), lambda i: (0, 0)),
        out_shape=jax.ShapeDtypeStruct((1, 2), jnp.float32),
        scratch_shapes=[pltpu.VMEM((1, 64), jnp.float32)],
    )(agg64, hs, dinv, w2, b2r, gf8, wp1a, wp1b, bp1r, wp2, bp2r)


def kernel(x, edge_index, W1, b1, W2, b2, Wp1, bp1, Wp2, bp2):
    n, f = x.shape
    e = edge_index.shape[1]
    n_pad = -(-n // 512) * 512
    h = W1.shape[1]
    assert h == 64 and f == 5

    e32 = edge_index.astype(jnp.int32)
    src, dst = e32[0], e32[1]

    x_pad = jnp.zeros((n_pad, 8), jnp.float32).at[:n, :f].set(x)
    w1p = jnp.zeros((8, h), jnp.float32).at[:f].set(W1)
    b1r = b1.reshape(1, h)
    b2r = b2.reshape(1, h)
    wp1a = Wp1[:h]
    wp1b = jnp.zeros((8, 32), jnp.float32).at[:6].set(Wp1[h:])
    bp1r = bp1.reshape(1, 32)
    bp2r = bp2.reshape(1, 2)
    z8 = jnp.zeros((512, 8), jnp.float32)
    zrows = jnp.zeros((256, 64), jnp.bfloat16)

    deg_parts = _sc_deg(dst, n_pad, e)
    dinv, xs8, gsums = _tc_prep(deg_parts, x_pad, n, n_pad)

    s0m, s1m, s2, s3, s4, cntm = (gsums[0, j] for j in range(6))
    avg_l = jnp.where(cntm > 0, s0m / jnp.maximum(cntm, 1.0), 0.0)
    avg_m = jnp.where(cntm > 0, s1m / jnp.maximum(cntm, 1.0), 0.0)
    gf8 = jnp.stack([s2, s3, s4, s3 + s4, avg_l, avg_m,
                     jnp.float32(0), jnp.float32(0)]).reshape(1, 8)

    agg8 = _sc_agg8(src, dst, xs8, z8, n_pad, e)
    hs = _tc_layer1(agg8, xs8, dinv, w1p, b1r, n_pad)
    agg64 = _sc_agg64(src, dst, hs, zrows, n_pad, e)
    return _tc_final(agg64, hs, dinv, W2, b2r, gf8, wp1a, wp1b, bp1r, Wp2,
                     bp2r, n, n_pad)
